# Initial kernel scaffold; baseline (speedup 1.0000x reference)
import jax
import jax.numpy as jnp
from jax.experimental import pallas as pl


def kernel(x, embedding):
    raise NotImplementedError("write your pallas kernel here")


# speedup vs baseline: 1.8459x; 1.8459x over previous
import jax
import jax.numpy as jnp
from jax import lax
from jax.experimental import pallas as pl
from jax.experimental.pallas import tpu as pltpu
from jax.experimental.pallas import tpu_sc as plsc

EMBED_DIM = 64
NUM_CORES = 2
NUM_SUBCORES = 16
NUM_WORKERS = NUM_CORES * NUM_SUBCORES
CHUNK = 1024


def _gather_body(x_hbm, table_hbm, out_hbm, idx_v, rows_v, sem):
    wid = lax.axis_index("s") * NUM_CORES + lax.axis_index("c")
    total = out_hbm.shape[0]
    per_w = total // NUM_WORKERS
    n_chunks = per_w // CHUNK
    base = wid * per_w

    def step(i, carry):
        off = base + i * CHUNK
        pltpu.sync_copy(x_hbm.at[pl.ds(off, CHUNK)], idx_v)
        pltpu.async_copy(table_hbm.at[idx_v], rows_v, sem).wait()
        pltpu.sync_copy(rows_v, out_hbm.at[pl.ds(off, CHUNK)])
        return carry

    lax.fori_loop(0, n_chunks, step, 0)


@jax.jit
def kernel(x, embedding):
    batch, hist = x.shape
    total = batch * hist
    xf = x.reshape(total)
    gather = pl.kernel(
        _gather_body,
        mesh=plsc.VectorSubcoreMesh(core_axis_name="c", subcore_axis_name="s"),
        out_type=jax.ShapeDtypeStruct((total, EMBED_DIM), jnp.float32),
        compiler_params=pltpu.CompilerParams(use_tc_tiling_on_sc=False),
        scratch_types=[
            pltpu.VMEM((CHUNK,), jnp.int32),
            pltpu.VMEM((CHUNK, EMBED_DIM), jnp.float32),
            pltpu.SemaphoreType.DMA,
        ],
    )
    out = gather(xf, embedding)
    return out.reshape(batch, hist, EMBED_DIM)


# speedup vs baseline: 1.8758x; 1.0162x over previous
import jax
import jax.numpy as jnp
from jax import lax
from jax.experimental import pallas as pl
from jax.experimental.pallas import tpu as pltpu
from jax.experimental.pallas import tpu_sc as plsc

EMBED_DIM = 64
NUM_CORES = 2
NUM_SUBCORES = 16
NUM_WORKERS = NUM_CORES * NUM_SUBCORES
CHUNK = 800


def _gather_body(x_hbm, table_hbm, out_hbm,
                 idx_v, rows0, rows1, gsem0, gsem1, wsem0, wsem1):
    wid = lax.axis_index("s") * NUM_CORES + lax.axis_index("c")
    total = out_hbm.shape[0]
    per_w = total // NUM_WORKERS
    n_chunks = per_w // CHUNK
    base = wid * per_w

    pltpu.sync_copy(x_hbm.at[pl.ds(base, per_w)], idx_v)

    rows = (rows0, rows1)
    gsems = (gsem0, gsem1)
    wsems = (wsem0, wsem1)

    def gather_start(c, b):
        pltpu.async_copy(
            table_hbm.at[idx_v.at[pl.ds(c * CHUNK, CHUNK)]], rows[b], gsems[b])

    def gather_wait(b):
        pltpu.make_async_copy(
            table_hbm.at[idx_v.at[pl.ds(0, CHUNK)]], rows[b], gsems[b]).wait()

    def write_start(c, b):
        pltpu.async_copy(
            rows[b], out_hbm.at[pl.ds(base + c * CHUNK, CHUNK)], wsems[b])

    def write_wait(b):
        pltpu.make_async_copy(
            rows[b], out_hbm.at[pl.ds(base, CHUNK)], wsems[b]).wait()

    gather_start(0, 0)
    gather_start(1, 1)

    def pair(g, carry):
        c = 2 * g
        for b in range(2):
            gather_wait(b)
            write_start(c + b, b)
            write_wait(b)
            gather_start(c + b + 2, b)
        return carry

    lax.fori_loop(0, n_chunks // 2 - 1, pair, 0)

    for b in range(2):
        gather_wait(b)
        write_start(n_chunks - 2 + b, b)
        write_wait(b)


@jax.jit
def kernel(x, embedding):
    batch, hist = x.shape
    total = batch * hist
    xf = x.reshape(total)
    gather = pl.kernel(
        _gather_body,
        mesh=plsc.VectorSubcoreMesh(core_axis_name="c", subcore_axis_name="s"),
        out_type=jax.ShapeDtypeStruct((total, EMBED_DIM), jnp.float32),
        compiler_params=pltpu.CompilerParams(use_tc_tiling_on_sc=False),
        scratch_types=[
            pltpu.VMEM((total // NUM_WORKERS,), jnp.int32),
            pltpu.VMEM((CHUNK, EMBED_DIM), jnp.float32),
            pltpu.VMEM((CHUNK, EMBED_DIM), jnp.float32),
            pltpu.SemaphoreType.DMA,
            pltpu.SemaphoreType.DMA,
            pltpu.SemaphoreType.DMA,
            pltpu.SemaphoreType.DMA,
        ],
    )
    out = gather(xf, embedding)
    return out.reshape(batch, hist, EMBED_DIM)


# speedup vs baseline: 1.8765x; 1.0004x over previous
import jax
import jax.numpy as jnp
from jax import lax
from jax.experimental import pallas as pl
from jax.experimental.pallas import tpu as pltpu
from jax.experimental.pallas import tpu_sc as plsc

EMBED_DIM = 64
NUM_CORES = 2
NUM_SUBCORES = 16
NUM_WORKERS = NUM_CORES * NUM_SUBCORES
CHUNK = 400
NBUF = 4


def _gather_body(x_hbm, table_hbm, out_hbm, idx_v, *bufs):
    rows = bufs[:NBUF]
    gsems = bufs[NBUF:2 * NBUF]
    wsems = bufs[2 * NBUF:]
    wid = lax.axis_index("s") * NUM_CORES + lax.axis_index("c")
    total = out_hbm.shape[0]
    per_w = total // NUM_WORKERS
    n_chunks = per_w // CHUNK
    base = wid * per_w

    pltpu.sync_copy(x_hbm.at[pl.ds(base, per_w)], idx_v)

    def gather_start(c, b):
        pltpu.async_copy(
            table_hbm.at[idx_v.at[pl.ds(c * CHUNK, CHUNK)]], rows[b], gsems[b])

    def gather_wait(b):
        pltpu.make_async_copy(
            table_hbm.at[idx_v.at[pl.ds(0, CHUNK)]], rows[b], gsems[b]).wait()

    def write_start(c, b):
        pltpu.async_copy(
            rows[b], out_hbm.at[pl.ds(base + c * CHUNK, CHUNK)], wsems[b])

    def write_wait(b):
        pltpu.make_async_copy(
            rows[b], out_hbm.at[pl.ds(base, CHUNK)], wsems[b]).wait()

    for b in range(NBUF):
        gather_start(b, b)

    def group(g, carry):
        c = NBUF * g
        for b in range(NBUF):
            gather_wait(b)
            write_start(c + b, b)
            write_wait(b)
            gather_start(c + b + NBUF, b)
        return carry

    lax.fori_loop(0, n_chunks // NBUF - 1, group, 0)

    for b in range(NBUF):
        gather_wait(b)
        write_start(n_chunks - NBUF + b, b)
        write_wait(b)


@jax.jit
def kernel(x, embedding):
    batch, hist = x.shape
    total = batch * hist
    xf = x.reshape(total)
    gather = pl.kernel(
        _gather_body,
        mesh=plsc.VectorSubcoreMesh(core_axis_name="c", subcore_axis_name="s"),
        out_type=jax.ShapeDtypeStruct((total, EMBED_DIM), jnp.float32),
        compiler_params=pltpu.CompilerParams(use_tc_tiling_on_sc=False),
        scratch_types=(
            [pltpu.VMEM((total // NUM_WORKERS,), jnp.int32)]
            + [pltpu.VMEM((CHUNK, EMBED_DIM), jnp.float32)] * NBUF
            + [pltpu.SemaphoreType.DMA] * (2 * NBUF)
        ),
    )
    out = gather(xf, embedding)
    return out.reshape(batch, hist, EMBED_DIM)
